# Initial kernel scaffold; baseline (speedup 1.0000x reference)
import jax
import jax.numpy as jnp
from jax.experimental import pallas as pl


def kernel(inp, indices, weights, weights_factors):
    raise NotImplementedError("write your pallas kernel here")


# speedup vs baseline: 4.2321x; 4.2321x over previous
import functools

import jax
import jax.numpy as jnp
from jax import lax
from jax.experimental import pallas as pl
from jax.experimental.pallas import tpu as pltpu
from jax.experimental.pallas import tpu_sc as plsc

N_NEURONS = 16384
N_BKG = 100
N_SYN_BASIS = 5
ROWS_PER_BAND = 32
NBANDS = N_NEURONS // ROWS_PER_BAND
SLOTS = ROWS_PER_BAND * N_BKG
BWORDS = SLOTS * N_SYN_BASIS
CH = 128
PAD = 256
BANDS_PER_WORKER = NBANDS // 32


def _scalar_from_vmem(vec_ref, idx, iot):
    off = (idx // 16) * 16
    win = vec_ref[pl.ds(off, 16)]
    rel = idx - off
    s = jnp.int32(0)
    for k in range(16):
        s = jnp.where(rel == k, win[k], s)
    return s


def _make_sc_scatter():
    mesh = plsc.VectorSubcoreMesh(core_axis_name="c", subcore_axis_name="s")

    @functools.partial(
        pl.kernel,
        mesh=mesh,
        compiler_params=pltpu.CompilerParams(
            needs_layout_passes=False, use_tc_tiling_on_sc=False),
        out_type=jax.ShapeDtypeStruct((NBANDS, BWORDS), jnp.float32),
        scratch_types=[
            pltpu.VMEM((CH,), jnp.int32),
            pltpu.VMEM((CH,), jnp.float32),
            pltpu.VMEM((CH * N_SYN_BASIS,), jnp.float32),
            pltpu.VMEM((NBANDS + 16,), jnp.int32),
            pltpu.VMEM((N_SYN_BASIS, CH), jnp.float32),
            pltpu.VMEM((N_SYN_BASIS, CH), jnp.int32),
            pltpu.VMEM((BWORDS,), jnp.float32),
            pltpu.VMEM_SHARED((16 * BWORDS,), jnp.float32),
        ],
    )
    def sc_scatter(p_hbm, w_hbm, f_hbm, bs_hbm, z_hbm, out_hbm,
                   p_v, w_v, f_v, bs_v, vals_v, slots_v, zer_v, acc_sh):
        cid = lax.axis_index("c")
        sid = lax.axis_index("s")
        wid = sid * 2 + cid
        tbase = sid * BWORDS
        iot = lax.iota(jnp.int32, 16)

        pltpu.sync_copy(bs_hbm, bs_v)
        pltpu.sync_copy(z_hbm, zer_v)

        for q in range(BANDS_PER_WORKER):
            band = wid * BANDS_PER_WORKER + q
            s = _scalar_from_vmem(bs_v, band, iot)
            e = _scalar_from_vmem(bs_v, band + 1, iot)
            pltpu.sync_copy(zer_v, acc_sh.at[pl.ds(tbase, BWORDS)])

            s_al = (s // 8) * 8
            nch = (e - s_al + (CH - 1)) // CH
            band_row0 = band * ROWS_PER_BAND

            def chunk_body(i, carry, s=s, e=e, s_al=s_al, band_row0=band_row0):
                cbase = s_al + i * CH
                pltpu.sync_copy(p_hbm.at[pl.ds(cbase, CH)], p_v)
                pltpu.sync_copy(w_hbm.at[pl.ds(cbase, CH)], w_v)
                pltpu.sync_copy(
                    f_hbm.at[pl.ds(cbase * N_SYN_BASIS, CH * N_SYN_BASIS)], f_v)
                for g in range(CH * N_SYN_BASIS // 16):
                    j0 = g * 16 + iot
                    n_loc = j0 // N_SYN_BASIS
                    r = j0 - n_loc * N_SYN_BASIS
                    nglob = cbase + n_loc
                    valid = (nglob >= s) & (nglob < e)
                    pg = plsc.load_gather(p_v, [n_loc])
                    wg = plsc.load_gather(w_v, [n_loc])
                    fv = f_v[pl.ds(g * 16, 16)]
                    val = jnp.where(valid, wg * fv, jnp.float32(0.0))
                    idx = jnp.where(
                        valid, (pg - band_row0) * N_SYN_BASIS + r,
                        jnp.int32(0)) + tbase
                    jr, jc = g // (CH // 16), (g % (CH // 16)) * 16
                    vals_v[jr, pl.ds(jc, 16)] = val
                    slots_v[jr, pl.ds(jc, 16)] = idx
                for jr in range(N_SYN_BASIS):
                    pltpu.sync_copy(vals_v.at[jr],
                                    acc_sh.at[slots_v.at[jr]], add=True)
                return carry

            lax.fori_loop(0, nch, chunk_body, jnp.int32(0))
            pltpu.sync_copy(acc_sh.at[pl.ds(tbase, BWORDS)], out_hbm.at[band])

    return sc_scatter


_sc_scatter = _make_sc_scatter()


GRP = 4


def _mm_body(s_ref, b_ref, o_ref):
    s = s_ref[...]
    parts = [jnp.dot(s, b_ref[q], preferred_element_type=jnp.float32)
             for q in range(GRP)]
    o_ref[...] = jnp.concatenate(parts, axis=-1)


def _band_matmul(spikes, bands):
    w = ROWS_PER_BAND * N_SYN_BASIS
    return pl.pallas_call(
        _mm_body,
        grid=(NBANDS // GRP,),
        in_specs=[
            pl.BlockSpec((spikes.shape[0], N_BKG), lambda i: (0, 0)),
            pl.BlockSpec((GRP, N_BKG, w), lambda i: (i, 0, 0)),
        ],
        out_specs=pl.BlockSpec((spikes.shape[0], GRP * w), lambda i: (0, i)),
        out_shape=jax.ShapeDtypeStruct(
            (spikes.shape[0], N_NEURONS * N_SYN_BASIS), jnp.float32),
    )(spikes, bands)


def kernel(inp, indices, weights, weights_factors):
    batch_size, seq_len, n_bkg = inp.shape
    spike_key = jax.random.key(42)
    spikes = (jax.random.uniform(spike_key, (batch_size, seq_len, n_bkg))
              < 250.0 * 0.001).astype(jnp.float32)
    spikes = spikes.reshape(batch_size * seq_len, n_bkg)

    rows = indices[:, 0].astype(jnp.int32)
    cols = indices[:, 1].astype(jnp.int32)
    p = cols * ROWS_PER_BAND + rows
    band_starts = jnp.searchsorted(
        rows, jnp.arange(NBANDS + 16, dtype=jnp.int32) * ROWS_PER_BAND
    ).astype(jnp.int32)

    p_pad = jnp.pad(p, (0, PAD))
    w_pad = jnp.pad(weights.astype(jnp.float32), (0, PAD))
    f_pad = jnp.pad(weights_factors.astype(jnp.float32).reshape(-1),
                    (0, PAD * N_SYN_BASIS))
    zeros_blk = jnp.zeros((BWORDS,), jnp.float32)

    bands = _sc_scatter(p_pad, w_pad, f_pad, band_starts, zeros_blk)
    bands = bands.reshape(NBANDS, N_BKG, ROWS_PER_BAND * N_SYN_BASIS)

    out = _band_matmul(spikes, bands)
    return out.reshape(batch_size, seq_len, -1)


# speedup vs baseline: 6.3993x; 1.5121x over previous
import functools

import jax
import jax.numpy as jnp
from jax import lax
from jax.experimental import pallas as pl
from jax.experimental.pallas import tpu as pltpu
from jax.experimental.pallas import tpu_sc as plsc

N_NEURONS = 16384
N_BKG = 100
N_SYN_BASIS = 5
NNZ = 163840
ROWS_PER_BAND = 128
NBANDS = N_NEURONS // ROWS_PER_BAND
COLS_PAD = 104
BAND_W = ROWS_PER_BAND * N_SYN_BASIS
BWORDS = N_BKG * BAND_W
OUT_BROW = COLS_PAD * BAND_W
PAD_W = OUT_BROW - BWORDS
CH = 128
NWORK = 30
SEARCH_STEPS = 18


def _make_sc_scatter():
    mesh = plsc.VectorSubcoreMesh(core_axis_name="c", subcore_axis_name="s")

    @functools.partial(
        pl.kernel,
        mesh=mesh,
        compiler_params=pltpu.CompilerParams(
            needs_layout_passes=False, use_tc_tiling_on_sc=False),
        out_type=jax.ShapeDtypeStruct((NBANDS * OUT_BROW,), jnp.float32),
        scratch_types=[
            pltpu.VMEM((CH,), jnp.int32),
            pltpu.VMEM((CH,), jnp.float32),
            pltpu.VMEM((CH * N_SYN_BASIS,), jnp.float32),
            pltpu.VMEM((N_SYN_BASIS, CH), jnp.float32),
            pltpu.VMEM((N_SYN_BASIS, CH), jnp.int32),
            pltpu.VMEM((16,), jnp.int32),
            pltpu.VMEM((BWORDS,), jnp.float32),
            pltpu.VMEM_SHARED((15 * BWORDS,), jnp.float32),
            pltpu.SemaphoreType.DMA,
            pltpu.SemaphoreType.DMA,
            pltpu.SemaphoreType.DMA,
        ],
    )
    def sc_scatter(p_hbm, w_hbm, f_hbm, rows_hbm, z_hbm, out_hbm,
                   p_v, w_v, f_v, vals_v, slots_v, srch_v, zer_v, acc_sh,
                   sem_in, sem_sc, sem_s):
        cid = lax.axis_index("c")
        sid = lax.axis_index("s")
        tbase = sid * BWORDS
        iot = lax.iota(jnp.int32, 16)

        @pl.when(sid < 15)
        def _worker():
            aid = sid * 2 + cid
            bstart = aid * 4 + jnp.minimum(aid, jnp.int32(8))

            pltpu.sync_copy(z_hbm, zer_v)

            thresh = (bstart + iot) * ROWS_PER_BAND
            lo = jnp.zeros((16,), jnp.int32)
            hi = jnp.full((16,), NNZ, jnp.int32)
            for _ in range(SEARCH_STEPS):
                mid = jnp.minimum((lo + hi) // 2, jnp.int32(NNZ - 1))
                pltpu.async_copy(rows_hbm.at[mid], srch_v, sem_s).wait()
                cond = (srch_v[...] < thresh) & (lo < hi)
                shrink = (srch_v[...] >= thresh) & (lo < hi)
                lo = jnp.where(cond, mid + 1, lo)
                hi = jnp.where(shrink, mid, hi)

            def do_band(q, lo=lo, bstart=bstart):
                band = bstart + q
                s = lo[q]
                e = lo[q + 1]
                pltpu.sync_copy(zer_v, acc_sh.at[pl.ds(tbase, BWORDS)])

                s_al = (s // 8) * 8
                nch = (e - s_al + (CH - 1)) // CH
                band_row0 = band * ROWS_PER_BAND

                def chunk_body(i, carry):
                    l0 = s_al + i * CH
                    cbase = jnp.minimum(l0, jnp.int32(NNZ - CH))
                    lo_b = jnp.maximum(s, l0)
                    h1 = pltpu.async_copy(
                        p_hbm.at[pl.ds(cbase, CH)], p_v, sem_in)
                    h2 = pltpu.async_copy(
                        w_hbm.at[pl.ds(cbase, CH)], w_v, sem_in)
                    h3 = pltpu.async_copy(
                        f_hbm.at[pl.ds(cbase * N_SYN_BASIS,
                                       CH * N_SYN_BASIS)],
                        f_v, sem_in)
                    h1.wait()
                    h2.wait()
                    h3.wait()
                    for g in range(CH * N_SYN_BASIS // 16):
                        j0 = g * 16 + iot
                        n_loc = j0 // N_SYN_BASIS
                        r = j0 - n_loc * N_SYN_BASIS
                        nglob = cbase + n_loc
                        valid = (nglob >= lo_b) & (nglob < e)
                        pg = plsc.load_gather(p_v, [n_loc])
                        wg = plsc.load_gather(w_v, [n_loc])
                        fv = f_v[pl.ds(g * 16, 16)]
                        val = jnp.where(valid, wg * fv, jnp.float32(0.0))
                        idx = jnp.where(
                            valid, (pg - band_row0) * N_SYN_BASIS + r,
                            jnp.int32(0)) + tbase
                        jr, jc = g // (CH // 16), (g % (CH // 16)) * 16
                        vals_v[jr, pl.ds(jc, 16)] = val
                        slots_v[jr, pl.ds(jc, 16)] = idx
                    hs = [
                        pltpu.async_copy(vals_v.at[jr],
                                         acc_sh.at[slots_v.at[jr]],
                                         sem_sc, add=True)
                        for jr in range(N_SYN_BASIS)
                    ]
                    for h in hs:
                        h.wait()
                    return carry

                lax.fori_loop(0, nch, chunk_body, jnp.int32(0))
                pltpu.sync_copy(
                    acc_sh.at[pl.ds(tbase, BWORDS)],
                    out_hbm.at[pl.ds(band * OUT_BROW, BWORDS)])
                pltpu.sync_copy(
                    zer_v.at[pl.ds(0, PAD_W)],
                    out_hbm.at[pl.ds(band * OUT_BROW + BWORDS, PAD_W)])

            for q in range(4):
                do_band(q)

            @pl.when(aid < 8)
            def _fifth():
                do_band(4)

    return sc_scatter


_sc_scatter = _make_sc_scatter()


def _mm_body(s_ref, b_ref, o_ref):
    o_ref[...] = jnp.dot(s_ref[...], b_ref[...],
                         preferred_element_type=jnp.float32)


def _band_matmul(spikes_pad, bands2):
    seq = spikes_pad.shape[0]
    return pl.pallas_call(
        _mm_body,
        grid=(NBANDS,),
        in_specs=[
            pl.BlockSpec((seq, COLS_PAD), lambda i: (0, 0)),
            pl.BlockSpec((COLS_PAD, BAND_W), lambda i: (i, 0)),
        ],
        out_specs=pl.BlockSpec((seq, BAND_W), lambda i: (0, i)),
        out_shape=jax.ShapeDtypeStruct(
            (seq, N_NEURONS * N_SYN_BASIS), jnp.float32),
    )(spikes_pad, bands2)


def kernel(inp, indices, weights, weights_factors):
    batch_size, seq_len, n_bkg = inp.shape
    spike_key = jax.random.key(42)
    spikes = (jax.random.uniform(spike_key, (batch_size, seq_len, n_bkg))
              < 250.0 * 0.001).astype(jnp.float32)
    spikes = spikes.reshape(batch_size * seq_len, n_bkg)
    spikes_pad = jnp.pad(spikes, ((0, 0), (0, COLS_PAD - n_bkg)))

    rows = indices[:, 0].astype(jnp.int32)
    cols = indices[:, 1].astype(jnp.int32)
    p = cols * ROWS_PER_BAND + rows
    w32 = weights.astype(jnp.float32)
    f_flat = weights_factors.astype(jnp.float32).reshape(-1)
    zeros_blk = jnp.zeros((BWORDS,), jnp.float32)

    bands = _sc_scatter(p, w32, f_flat, rows, zeros_blk)
    bands2 = bands.reshape(NBANDS * COLS_PAD, BAND_W)

    out = _band_matmul(spikes_pad, bands2)
    return out.reshape(batch_size, seq_len, -1)


# speedup vs baseline: 9.2884x; 1.4515x over previous
import functools

import jax
import jax.numpy as jnp
from jax import lax
from jax.experimental import pallas as pl
from jax.experimental.pallas import tpu as pltpu
from jax.experimental.pallas import tpu_sc as plsc

N_NEURONS = 16384
N_BKG = 100
N_SYN_BASIS = 5
NNZ = 163840
ROWS_PER_BAND = 128
NBANDS = N_NEURONS // ROWS_PER_BAND
COLS_PAD = 104
BAND_W = ROWS_PER_BAND * N_SYN_BASIS
BWORDS = N_BKG * BAND_W
OUT_BROW = COLS_PAD * BAND_W
PAD_W = OUT_BROW - BWORDS
CH = 256
SEARCH_STEPS = 18
NROWS_SC = CH // 128


def _make_sc_scatter():
    mesh = plsc.VectorSubcoreMesh(core_axis_name="c", subcore_axis_name="s")

    @functools.partial(
        pl.kernel,
        mesh=mesh,
        compiler_params=pltpu.CompilerParams(
            needs_layout_passes=False, use_tc_tiling_on_sc=False),
        out_type=jax.ShapeDtypeStruct((NBANDS * OUT_BROW,), jnp.float32),
        scratch_types=[
            pltpu.VMEM((CH,), jnp.int32),
            pltpu.VMEM((CH,), jnp.float32),
            pltpu.VMEM((N_SYN_BASIS, CH), jnp.float32),
            pltpu.VMEM((N_SYN_BASIS * NROWS_SC, 128), jnp.float32),
            pltpu.VMEM((N_SYN_BASIS * NROWS_SC, 128), jnp.int32),
            pltpu.VMEM((16,), jnp.int32),
            pltpu.VMEM((BWORDS,), jnp.float32),
            pltpu.VMEM_SHARED((15 * BWORDS,), jnp.float32),
            pltpu.SemaphoreType.DMA,
            pltpu.SemaphoreType.DMA,
            pltpu.SemaphoreType.DMA,
        ],
    )
    def sc_scatter(p_hbm, w_hbm, ft_hbm, rows_hbm, z_hbm, out_hbm,
                   p_v, w_v, f_v, vals_v, slots_v, srch_v, zer_v, acc_sh,
                   sem_in, sem_sc, sem_s):
        cid = lax.axis_index("c")
        sid = lax.axis_index("s")
        tbase = sid * BWORDS
        iot = lax.iota(jnp.int32, 16)

        @pl.when(sid < 15)
        def _worker():
            aid = sid * 2 + cid
            bstart = aid * 4 + jnp.minimum(aid, jnp.int32(8))

            pltpu.sync_copy(z_hbm, zer_v)

            thresh = (bstart + iot) * ROWS_PER_BAND
            lo = jnp.zeros((16,), jnp.int32)
            hi = jnp.full((16,), NNZ, jnp.int32)
            for _ in range(SEARCH_STEPS):
                mid = jnp.minimum((lo + hi) // 2, jnp.int32(NNZ - 1))
                pltpu.async_copy(rows_hbm.at[mid], srch_v, sem_s).wait()
                cond = (srch_v[...] < thresh) & (lo < hi)
                shrink = (srch_v[...] >= thresh) & (lo < hi)
                lo = jnp.where(cond, mid + 1, lo)
                hi = jnp.where(shrink, mid, hi)

            def do_band(q, lo=lo, bstart=bstart):
                band = bstart + q
                s = lo[q]
                e = lo[q + 1]
                pltpu.sync_copy(zer_v, acc_sh.at[pl.ds(tbase, BWORDS)])

                s_al = (s // 8) * 8
                nch = (e - s_al + (CH - 1)) // CH
                band_row0 = band * ROWS_PER_BAND

                def chunk_body(i, carry):
                    l0 = s_al + i * CH
                    cbase = jnp.minimum(l0, jnp.int32(NNZ - CH))
                    lo_b = jnp.maximum(s, l0)
                    hin = [
                        pltpu.async_copy(
                            p_hbm.at[pl.ds(cbase, CH)], p_v, sem_in),
                        pltpu.async_copy(
                            w_hbm.at[pl.ds(cbase, CH)], w_v, sem_in),
                    ]
                    hin += [
                        pltpu.async_copy(
                            ft_hbm.at[pl.ds(r * NNZ + cbase, CH)],
                            f_v.at[r], sem_in)
                        for r in range(N_SYN_BASIS)
                    ]
                    for h in hin:
                        h.wait()
                    for g in range(CH // 16):
                        g16 = pl.ds(g * 16, 16)
                        nglob = cbase + g * 16 + iot
                        valid = (nglob >= lo_b) & (nglob < e)
                        qm5 = (p_v[g16] - band_row0) * N_SYN_BASIS
                        qm5 = jnp.where(valid, qm5, jnp.int32(0)) + tbase
                        wv = w_v[g16]
                        jr, jc = g // 8, pl.ds((g % 8) * 16, 16)
                        for r in range(N_SYN_BASIS):
                            val = jnp.where(valid, wv * f_v[r, g16],
                                            jnp.float32(0.0))
                            vals_v[r * NROWS_SC + jr, jc] = val
                            slots_v[r * NROWS_SC + jr, jc] = qm5 + r
                    hs = [
                        pltpu.async_copy(vals_v.at[jr],
                                         acc_sh.at[slots_v.at[jr]],
                                         sem_sc, add=True)
                        for jr in range(N_SYN_BASIS * NROWS_SC)
                    ]
                    for h in hs:
                        h.wait()
                    return carry

                lax.fori_loop(0, nch, chunk_body, jnp.int32(0))
                pltpu.sync_copy(
                    acc_sh.at[pl.ds(tbase, BWORDS)],
                    out_hbm.at[pl.ds(band * OUT_BROW, BWORDS)])
                pltpu.sync_copy(
                    zer_v.at[pl.ds(0, PAD_W)],
                    out_hbm.at[pl.ds(band * OUT_BROW + BWORDS, PAD_W)])

            for q in range(4):
                do_band(q)

            @pl.when(aid < 8)
            def _fifth():
                do_band(4)

    return sc_scatter


_sc_scatter = _make_sc_scatter()


def _mm_body(s_ref, b_ref, o_ref):
    o_ref[...] = jnp.dot(s_ref[...], b_ref[...],
                         preferred_element_type=jnp.float32)


def _band_matmul(spikes_pad, bands2):
    seq = spikes_pad.shape[0]
    return pl.pallas_call(
        _mm_body,
        grid=(NBANDS,),
        in_specs=[
            pl.BlockSpec((seq, COLS_PAD), lambda i: (0, 0)),
            pl.BlockSpec((COLS_PAD, BAND_W), lambda i: (i, 0)),
        ],
        out_specs=pl.BlockSpec((seq, BAND_W), lambda i: (0, i)),
        out_shape=jax.ShapeDtypeStruct(
            (seq, N_NEURONS * N_SYN_BASIS), jnp.float32),
    )(spikes_pad, bands2)


def kernel(inp, indices, weights, weights_factors):
    batch_size, seq_len, n_bkg = inp.shape
    spike_key = jax.random.key(42)
    spikes = (jax.random.uniform(spike_key, (batch_size, seq_len, n_bkg))
              < 250.0 * 0.001).astype(jnp.float32)
    spikes = spikes.reshape(batch_size * seq_len, n_bkg)
    spikes_pad = jnp.pad(spikes, ((0, 0), (0, COLS_PAD - n_bkg)))

    rows = indices[:, 0].astype(jnp.int32)
    cols = indices[:, 1].astype(jnp.int32)
    p = cols * ROWS_PER_BAND + rows
    w32 = weights.astype(jnp.float32)
    ft = weights_factors.astype(jnp.float32).T.reshape(-1)
    zeros_blk = jnp.zeros((BWORDS,), jnp.float32)

    bands = _sc_scatter(p, w32, ft, rows, zeros_blk)
    bands2 = bands.reshape(NBANDS * COLS_PAD, BAND_W)

    out = _band_matmul(spikes_pad, bands2)
    return out.reshape(batch_size, seq_len, -1)


# speedup vs baseline: 11.6159x; 1.2506x over previous
import functools

import jax
import jax.numpy as jnp
from jax import lax
from jax.experimental import pallas as pl
from jax.experimental.pallas import tpu as pltpu
from jax.experimental.pallas import tpu_sc as plsc

N_NEURONS = 16384
N_BKG = 100
N_SYN_BASIS = 5
NNZ = 163840
ROWS_PER_BAND = 128
NBANDS = N_NEURONS // ROWS_PER_BAND
COLS_PAD = 104
BAND_W = ROWS_PER_BAND * N_SYN_BASIS
BWORDS = N_BKG * BAND_W
OUT_BROW = COLS_PAD * BAND_W
PAD_W = OUT_BROW - BWORDS
CH = 256
SEARCH_STEPS = 18
NROWS_SC = CH // 128


def _make_sc_scatter():
    mesh = plsc.VectorSubcoreMesh(core_axis_name="c", subcore_axis_name="s")

    @functools.partial(
        pl.kernel,
        mesh=mesh,
        compiler_params=pltpu.CompilerParams(
            needs_layout_passes=False, use_tc_tiling_on_sc=False),
        out_type=jax.ShapeDtypeStruct((NBANDS * OUT_BROW,), jnp.float32),
        scratch_types=[
            pltpu.VMEM((CH,), jnp.int32),
            pltpu.VMEM((CH,), jnp.float32),
            pltpu.VMEM((N_SYN_BASIS, CH), jnp.float32),
            pltpu.VMEM((N_SYN_BASIS * NROWS_SC, 128), jnp.float32),
            pltpu.VMEM((N_SYN_BASIS * NROWS_SC, 128), jnp.int32),
            pltpu.VMEM((16,), jnp.int32),
            pltpu.VMEM((BWORDS,), jnp.float32),
            pltpu.VMEM_SHARED((15 * BWORDS,), jnp.float32),
            pltpu.SemaphoreType.DMA,
            pltpu.SemaphoreType.DMA,
            pltpu.SemaphoreType.DMA,
        ],
    )
    def sc_scatter(p_hbm, w_hbm, ft_hbm, rows_hbm, z_hbm, out_hbm,
                   p_v, w_v, f_v, vals_v, slots_v, srch_v, zer_v, acc_sh,
                   sem_in, sem_sc, sem_s):
        cid = lax.axis_index("c")
        sid = lax.axis_index("s")
        tbase = sid * BWORDS
        iot = lax.iota(jnp.int32, 16)

        @pl.when(sid < 15)
        def _worker():
            aid = sid * 2 + cid
            bstart = aid * 4 + jnp.minimum(aid, jnp.int32(8))

            pltpu.sync_copy(z_hbm, zer_v)

            thresh = (bstart + iot) * ROWS_PER_BAND
            lo = jnp.zeros((16,), jnp.int32)
            hi = jnp.full((16,), NNZ, jnp.int32)
            for _ in range(SEARCH_STEPS):
                mid = jnp.minimum((lo + hi) // 2, jnp.int32(NNZ - 1))
                pltpu.async_copy(rows_hbm.at[mid], srch_v, sem_s).wait()
                cond = (srch_v[...] < thresh) & (lo < hi)
                shrink = (srch_v[...] >= thresh) & (lo < hi)
                lo = jnp.where(cond, mid + 1, lo)
                hi = jnp.where(shrink, mid, hi)

            def do_band(q, lo=lo, bstart=bstart):
                band = bstart + q
                s = lo[q]
                e = lo[q + 1]
                pltpu.sync_copy(zer_v, acc_sh.at[pl.ds(tbase, BWORDS)])

                s_al = (s // 8) * 8
                nch = (e - s_al + (CH - 1)) // CH
                band_row0 = band * ROWS_PER_BAND

                def chunk_body(i, carry):
                    l0 = s_al + i * CH
                    cbase = jnp.minimum(l0, jnp.int32(NNZ - CH))
                    lo_b = jnp.maximum(s, l0)
                    hin = [
                        pltpu.async_copy(
                            p_hbm.at[pl.ds(cbase, CH)], p_v, sem_in),
                        pltpu.async_copy(
                            w_hbm.at[pl.ds(cbase, CH)], w_v, sem_in),
                    ]
                    hin += [
                        pltpu.async_copy(
                            ft_hbm.at[pl.ds(r * NNZ + cbase, CH)],
                            f_v.at[r], sem_in)
                        for r in range(N_SYN_BASIS)
                    ]
                    for h in hin:
                        h.wait()
                    for g in range(CH // 16):
                        g16 = pl.ds(g * 16, 16)
                        nglob = cbase + g * 16 + iot
                        valid = (nglob >= lo_b) & (nglob < e)
                        qm5 = (p_v[g16] - band_row0) * N_SYN_BASIS
                        qm5 = jnp.where(valid, qm5, jnp.int32(0)) + tbase
                        wv = w_v[g16]
                        jr, jc = g // 8, pl.ds((g % 8) * 16, 16)
                        for r in range(N_SYN_BASIS):
                            val = jnp.where(valid, wv * f_v[r, g16],
                                            jnp.float32(0.0))
                            vals_v[r * NROWS_SC + jr, jc] = val
                            slots_v[r * NROWS_SC + jr, jc] = qm5 + r
                    hs = [
                        pltpu.async_copy(vals_v.at[jr],
                                         acc_sh.at[slots_v.at[jr]],
                                         sem_sc, add=True)
                        for jr in range(N_SYN_BASIS * NROWS_SC)
                    ]
                    for h in hs:
                        h.wait()
                    return carry

                lax.fori_loop(0, nch, chunk_body, jnp.int32(0))
                pltpu.sync_copy(
                    acc_sh.at[pl.ds(tbase, BWORDS)],
                    out_hbm.at[pl.ds(band * OUT_BROW, BWORDS)])
                pltpu.sync_copy(
                    zer_v.at[pl.ds(0, PAD_W)],
                    out_hbm.at[pl.ds(band * OUT_BROW + BWORDS, PAD_W)])

            for q in range(4):
                do_band(q)

            @pl.when(aid < 8)
            def _fifth():
                do_band(4)

    return sc_scatter


_sc_scatter = _make_sc_scatter()


GRP = 4


def _mm_body(s_ref, b_ref, o_ref):
    s = s_ref[...]
    parts = []
    for q in range(GRP):
        bq = b_ref[pl.ds(q * COLS_PAD, COLS_PAD), :]
        parts.append(jnp.dot(s, bq, preferred_element_type=jnp.float32))
    o_ref[...] = jnp.concatenate(parts, axis=-1)


def _band_matmul(spikes_pad, bands2):
    seq = spikes_pad.shape[0]
    return pl.pallas_call(
        _mm_body,
        grid=(NBANDS // GRP,),
        in_specs=[
            pl.BlockSpec((seq, COLS_PAD), lambda i: (0, 0)),
            pl.BlockSpec((GRP * COLS_PAD, BAND_W), lambda i: (i, 0)),
        ],
        out_specs=pl.BlockSpec((seq, GRP * BAND_W), lambda i: (0, i)),
        out_shape=jax.ShapeDtypeStruct(
            (seq, N_NEURONS * N_SYN_BASIS), jnp.float32),
    )(spikes_pad, bands2)


def kernel(inp, indices, weights, weights_factors):
    batch_size, seq_len, n_bkg = inp.shape
    spike_key = jax.random.key(42)
    spikes = (jax.random.uniform(spike_key, (batch_size, seq_len, n_bkg))
              < 250.0 * 0.001).astype(jnp.float32)
    spikes = spikes.reshape(batch_size * seq_len, n_bkg)
    spikes_pad = jnp.pad(spikes, ((0, 0), (0, COLS_PAD - n_bkg)))

    rows = indices[:, 0].astype(jnp.int32)
    cols = indices[:, 1].astype(jnp.int32)
    p = cols * ROWS_PER_BAND + rows
    w32 = weights.astype(jnp.float32)
    ft = weights_factors.astype(jnp.float32).T.reshape(-1)
    zeros_blk = jnp.zeros((BWORDS,), jnp.float32)

    bands = _sc_scatter(p, w32, ft, rows, zeros_blk)
    bands2 = bands.reshape(NBANDS * COLS_PAD, BAND_W)

    out = _band_matmul(spikes_pad, bands2)
    return out.reshape(batch_size, seq_len, -1)
